# Initial kernel scaffold; baseline (speedup 1.0000x reference)
import jax
import jax.numpy as jnp
from jax.experimental import pallas as pl


def kernel(hidden_states, Wq, Wk, Wv, Wo, q_norm_w, k_norm_w, ln1_w, ln2_w, Wr, Wg, Wu, Wd, Wsg, Wsu, Wsd, position_ids):
    raise NotImplementedError("write your pallas kernel here")


# speedup vs baseline: 1.4256x; 1.4256x over previous
import math

import jax
import jax.numpy as jnp
from jax import lax
from jax.experimental import pallas as pl

HID = 1024
NH = 16
NKV = 4
HD = 64
E = 8
NG = 4
FF = 512
SFF = 2048
THETA = 8000000.0
EPS = 1e-05
SCALE = 1.0
BT = 256

F32 = jnp.float32


def _rope_tables(pos_f, bt):
    io = lax.broadcasted_iota(jnp.int32, (1, HD // 2), 1).astype(F32)
    inv = jnp.exp(io * (-2.0 * math.log(THETA) / HD))
    ang = pos_f * inv
    return jnp.cos(ang), jnp.sin(ang)


def _k1_body(pos_ref, hs_ref, wqkv_ref, ln1_ref, qnw_ref, knw_ref,
             q_ref, k_ref, v_ref):
    x = hs_ref[...]
    ms = jnp.mean(x * x, axis=1, keepdims=True)
    h = x * lax.rsqrt(ms + EPS) * ln1_ref[...]
    qkv = jnp.dot(h, wqkv_ref[...], preferred_element_type=F32)
    pos_f = pos_ref[...].astype(F32)
    cos, sin = _rope_tables(pos_f, x.shape[0])

    def norm_rope(mat, nheads, nw):
        outs = []
        for c in range(nheads):
            ch = mat[:, c * HD:(c + 1) * HD]
            m2 = jnp.mean(ch * ch, axis=1, keepdims=True)
            ch = ch * lax.rsqrt(m2 + EPS) * nw
            x1 = ch[:, :HD // 2]
            x2 = ch[:, HD // 2:]
            r = jnp.concatenate(
                [x1 * cos - x2 * sin, x1 * sin + x2 * cos], axis=1)
            outs.append(r[None, :, :])
        return jnp.concatenate(outs, axis=0)

    q_ref[...] = norm_rope(qkv[:, :NH * HD], NH, qnw_ref[...])
    k_ref[...] = norm_rope(qkv[:, NH * HD:(NH + NKV) * HD], NKV, knw_ref[...])
    vv = qkv[:, (NH + NKV) * HD:]
    v_ref[...] = jnp.concatenate(
        [vv[None, :, c * HD:(c + 1) * HD] for c in range(NKV)], axis=0)


def _k2_body(q_ref, k_ref, v_ref, o_ref):
    i = pl.program_id(1)
    bt = q_ref.shape[1]
    s_len = k_ref.shape[1]
    qb = q_ref[0]
    kb = k_ref[0]
    s = lax.dot_general(qb, kb, (((1,), (1,)), ((), ())),
                        preferred_element_type=F32)
    s = s * (1.0 / math.sqrt(float(HD)))
    row = lax.broadcasted_iota(jnp.int32, (bt, s_len), 0) + i * bt
    col = lax.broadcasted_iota(jnp.int32, (bt, s_len), 1)
    s = jnp.where(col <= row, s, -1e9)
    m = jnp.max(s, axis=1, keepdims=True)
    p = jnp.exp(s - m)
    p = p / jnp.sum(p, axis=1, keepdims=True)
    o_ref[0] = jnp.dot(p, v_ref[0], preferred_element_type=F32)


def _k3_body(ctx_ref, hid_ref, wo_ref, ln2_ref, wr_ref,
             res_ref, x_ref, log_ref):
    cc = jnp.concatenate([ctx_ref[h] for h in range(NH)], axis=1)
    a = hid_ref[...] + jnp.dot(cc, wo_ref[...],
                               preferred_element_type=F32)
    res_ref[...] = a
    ms = jnp.mean(a * a, axis=1, keepdims=True)
    xx = a * lax.rsqrt(ms + EPS) * ln2_ref[...]
    x_ref[...] = xx
    log_ref[...] = jnp.dot(xx, wr_ref[...], preferred_element_type=F32)


def _k4_body(log_ref, comb_ref):
    t = log_ref.shape[0]
    s = jax.nn.sigmoid(log_ref[...].astype(F32))
    gs = jnp.concatenate(
        [s[:, 2 * g:2 * g + 1] + s[:, 2 * g + 1:2 * g + 2] for g in range(NG)],
        axis=1)
    io4 = lax.broadcasted_iota(jnp.int32, (t, NG), 1)
    m1 = jnp.max(gs, axis=1, keepdims=True)
    a1 = jnp.min(jnp.where(gs == m1, io4, NG + 9), axis=1, keepdims=True)
    gs2 = jnp.where(io4 == a1, -1e30, gs)
    m2 = jnp.max(gs2, axis=1, keepdims=True)
    a2 = jnp.min(jnp.where(gs2 == m2, io4, NG + 9), axis=1, keepdims=True)
    io8 = lax.broadcasted_iota(jnp.int32, (t, E), 1)
    gid = io8 // (E // NG)
    sel = (gid == a1) | (gid == a2)
    masked = jnp.where(sel, s, 0.0)
    v1 = jnp.max(masked, axis=1, keepdims=True)
    i1 = jnp.min(jnp.where(masked == v1, io8, E + 9), axis=1, keepdims=True)
    masked2 = jnp.where(io8 == i1, -1.0, masked)
    v2 = jnp.max(masked2, axis=1, keepdims=True)
    i2 = jnp.min(jnp.where(masked2 == v2, io8, E + 9), axis=1, keepdims=True)
    tot = v1 + v2 + 1e-20
    w1 = v1 / tot * SCALE
    w2 = v2 / tot * SCALE
    comb_ref[...] = (jnp.where(io8 == i1, w1, 0.0)
                     + jnp.where(io8 == i2, w2, 0.0))


def _k5_body(x_ref, comb_ref, wg_ref, wu_ref, wd_ref, out_ref):
    e = pl.program_id(0)
    x = x_ref[...]
    g = jnp.dot(x, wg_ref[0], preferred_element_type=F32)
    u = jnp.dot(x, wu_ref[0], preferred_element_type=F32)
    hh = (g * jax.nn.sigmoid(g)) * u
    y = jnp.dot(hh, wd_ref[0], preferred_element_type=F32)
    io8 = lax.broadcasted_iota(jnp.int32, comb_ref.shape, 1)
    w = jnp.sum(jnp.where(io8 == e, comb_ref[...], 0.0), axis=1, keepdims=True)
    contrib = y * w

    @pl.when(e == 0)
    def _():
        out_ref[...] = contrib

    @pl.when(e > 0)
    def _():
        out_ref[...] = out_ref[...] + contrib


def _k6_body(x_ref, res_ref, moe_ref, wsg_ref, wsu_ref, wsd_ref, out_ref):
    x = x_ref[...]
    g = jnp.dot(x, wsg_ref[...], preferred_element_type=F32)
    u = jnp.dot(x, wsu_ref[...], preferred_element_type=F32)
    hh = (g * jax.nn.sigmoid(g)) * u
    y = jnp.dot(hh, wsd_ref[...], preferred_element_type=F32)
    out_ref[...] = res_ref[...] + moe_ref[...] + y


def kernel(hidden_states, Wq, Wk, Wv, Wo, q_norm_w, k_norm_w, ln1_w, ln2_w,
           Wr, Wg, Wu, Wd, Wsg, Wsu, Wsd, position_ids):
    B, S, D = hidden_states.shape
    T = B * S
    nb = T // BT
    hs = hidden_states.reshape(T, D)
    pos = position_ids.reshape(T, 1)

    perm = jnp.concatenate([jnp.arange(0, HD, 2), jnp.arange(1, HD, 2)])
    Wq_p = Wq.reshape(D, NH, HD)[:, :, perm].reshape(D, NH * HD)
    Wk_p = Wk.reshape(D, NKV, HD)[:, :, perm].reshape(D, NKV * HD)
    qnw = q_norm_w[perm].reshape(1, HD)
    knw = k_norm_w[perm].reshape(1, HD)
    wqkv = jnp.concatenate([Wq_p, Wk_p, Wv], axis=1)

    q, k, v = pl.pallas_call(
        _k1_body,
        grid=(nb,),
        in_specs=[
            pl.BlockSpec((BT, 1), lambda i: (i, 0)),
            pl.BlockSpec((BT, D), lambda i: (i, 0)),
            pl.BlockSpec((D, (NH + 2 * NKV) * HD), lambda i: (0, 0)),
            pl.BlockSpec((1, D), lambda i: (0, 0)),
            pl.BlockSpec((1, HD), lambda i: (0, 0)),
            pl.BlockSpec((1, HD), lambda i: (0, 0)),
        ],
        out_specs=[
            pl.BlockSpec((NH, BT, HD), lambda i: (0, i, 0)),
            pl.BlockSpec((NKV, BT, HD), lambda i: (0, i, 0)),
            pl.BlockSpec((NKV, BT, HD), lambda i: (0, i, 0)),
        ],
        out_shape=[
            jax.ShapeDtypeStruct((NH, T, HD), F32),
            jax.ShapeDtypeStruct((NKV, T, HD), F32),
            jax.ShapeDtypeStruct((NKV, T, HD), F32),
        ],
    )(pos, hs, wqkv, ln1_w.reshape(1, D), qnw, knw)

    rep = NH // NKV
    ctx = pl.pallas_call(
        _k2_body,
        grid=(NH, nb),
        in_specs=[
            pl.BlockSpec((1, BT, HD), lambda h, i: (h, i, 0)),
            pl.BlockSpec((1, T, HD), lambda h, i: (h // rep, 0, 0)),
            pl.BlockSpec((1, T, HD), lambda h, i: (h // rep, 0, 0)),
        ],
        out_specs=pl.BlockSpec((1, BT, HD), lambda h, i: (h, i, 0)),
        out_shape=jax.ShapeDtypeStruct((NH, T, HD), F32),
    )(q, k, v)

    attn_res, x, logits = pl.pallas_call(
        _k3_body,
        grid=(nb,),
        in_specs=[
            pl.BlockSpec((NH, BT, HD), lambda i: (0, i, 0)),
            pl.BlockSpec((BT, D), lambda i: (i, 0)),
            pl.BlockSpec((NH * HD, D), lambda i: (0, 0)),
            pl.BlockSpec((1, D), lambda i: (0, 0)),
            pl.BlockSpec((D, E), lambda i: (0, 0)),
        ],
        out_specs=[
            pl.BlockSpec((BT, D), lambda i: (i, 0)),
            pl.BlockSpec((BT, D), lambda i: (i, 0)),
            pl.BlockSpec((BT, E), lambda i: (i, 0)),
        ],
        out_shape=[
            jax.ShapeDtypeStruct((T, D), F32),
            jax.ShapeDtypeStruct((T, D), F32),
            jax.ShapeDtypeStruct((T, E), F32),
        ],
    )(ctx, hs, Wo, ln2_w.reshape(1, D), Wr)

    combine = pl.pallas_call(
        _k4_body,
        out_shape=jax.ShapeDtypeStruct((T, E), F32),
    )(logits)

    moe = pl.pallas_call(
        _k5_body,
        grid=(E,),
        in_specs=[
            pl.BlockSpec((T, D), lambda e: (0, 0)),
            pl.BlockSpec((T, E), lambda e: (0, 0)),
            pl.BlockSpec((1, D, FF), lambda e: (e, 0, 0)),
            pl.BlockSpec((1, D, FF), lambda e: (e, 0, 0)),
            pl.BlockSpec((1, FF, D), lambda e: (e, 0, 0)),
        ],
        out_specs=pl.BlockSpec((T, D), lambda e: (0, 0)),
        out_shape=jax.ShapeDtypeStruct((T, D), F32),
    )(x, combine, Wg, Wu, Wd)

    out = pl.pallas_call(
        _k6_body,
        grid=(nb,),
        in_specs=[
            pl.BlockSpec((BT, D), lambda i: (i, 0)),
            pl.BlockSpec((BT, D), lambda i: (i, 0)),
            pl.BlockSpec((BT, D), lambda i: (i, 0)),
            pl.BlockSpec((D, SFF), lambda i: (0, 0)),
            pl.BlockSpec((D, SFF), lambda i: (0, 0)),
            pl.BlockSpec((SFF, D), lambda i: (0, 0)),
        ],
        out_specs=pl.BlockSpec((BT, D), lambda i: (i, 0)),
        out_shape=jax.ShapeDtypeStruct((T, D), F32),
    )(x, attn_res, moe, Wsg, Wsu, Wsd)

    return out.reshape(B, S, D)
